# baseline (speedup 1.0000x reference)
import functools

import jax
import jax.numpy as jnp
from jax import lax
from jax.experimental import pallas as pl
from jax.experimental.pallas import tpu as pltpu
from jax.experimental.pallas import tpu_sc as plsc

N = 10000
E = 320000
NB = 256
H = 64
NC, NS = 2, 16
NW = NC * NS
EPAD = 327680
EPT = EPAD // NW
CHUNK = 64
NG = EPT // CHUNK
NBUF = 8
RPT = N // NS
NEG = -1.0e30

_NODE_BLK = 2000
_EDGE_BLK = 4096


def _node_init_body(nf, wn, bn, wmh, h_out, hp_out):
    hv = jnp.maximum(jnp.dot(nf[...], wn[...]) + bn[...], 0.0)
    h_out[...] = hv
    hp_out[...] = jnp.dot(hv, wmh[...])


def _edge_pre_body(ef, we, be, wme, bm, c_out):
    i = pl.program_id(0)
    ev = jnp.maximum(jnp.dot(ef[...], we[...]) + be[...], 0.0)
    cv = jnp.dot(ev, wme[...]) + bm[...]
    rows = i * _EDGE_BLK + lax.broadcasted_iota(jnp.int32, (_EDGE_BLK, 1), 0)
    c_out[...] = jnp.where(rows < E, cv, NEG)


def _update_body(h, agg, wuh, wua, bu, wmh, h_out, hp_out):
    a = agg[0] + agg[1]
    hv = jnp.maximum(
        jnp.dot(h[...], wuh[...]) + jnp.dot(a, wua[...]) + bu[...], 0.0)
    h_out[...] = hv
    hp_out[...] = jnp.dot(hv, wmh[...])


def _final_body(h, bidx, arep, w1h, w1a, b1, w2, b2, w3, b3, w4, b4, out):
    onehot = (bidx[...] == lax.broadcasted_iota(
        jnp.int32, (_NODE_BLK, NB), 1)).astype(jnp.float32)
    a = jnp.dot(onehot, arep[...])
    x = jnp.maximum(jnp.dot(h[...], w1h[...]) + jnp.dot(a, w1a[...]) + b1[...], 0.0)
    x = jnp.maximum(jnp.dot(x, w2[...]) + b2[...], 0.0)
    x = jnp.maximum(jnp.dot(x, w3[...]) + b3[...], 0.0)
    logit = jnp.dot(x, w4[...]) + b4[...]
    out[...] = (logit >= 0.0).astype(jnp.float32)


def _full(shape):
    return pl.BlockSpec(shape, lambda i: tuple(0 for _ in shape))


def _node_init(nf, wn, bn, wmh):
    return pl.pallas_call(
        _node_init_body,
        grid=(N // _NODE_BLK,),
        in_specs=[
            pl.BlockSpec((_NODE_BLK, 128), lambda i: (i, 0)),
            _full((128, H)), _full((1, H)), _full((H, H)),
        ],
        out_specs=[pl.BlockSpec((_NODE_BLK, H), lambda i: (i, 0))] * 2,
        out_shape=[jax.ShapeDtypeStruct((N, H), jnp.float32)] * 2,
    )(nf, wn, bn, wmh)


def _edge_pre(ef, we, be, wme, bm):
    return pl.pallas_call(
        _edge_pre_body,
        grid=(EPAD // _EDGE_BLK,),
        in_specs=[
            pl.BlockSpec((_EDGE_BLK, 16), lambda i: (i, 0)),
            _full((16, 32)), _full((1, 32)), _full((32, H)), _full((1, H)),
        ],
        out_specs=pl.BlockSpec((_EDGE_BLK, H), lambda i: (i, 0)),
        out_shape=jax.ShapeDtypeStruct((EPAD, H), jnp.float32),
    )(ef, we, be, wme, bm)


def _update(h, agg, wuh, wua, bu, wmh):
    return pl.pallas_call(
        _update_body,
        grid=(N // _NODE_BLK,),
        in_specs=[
            pl.BlockSpec((_NODE_BLK, H), lambda i: (i, 0)),
            pl.BlockSpec((NC, _NODE_BLK, H), lambda i: (0, i, 0)),
            _full((H, H)), _full((H, H)), _full((1, H)), _full((H, H)),
        ],
        out_specs=[pl.BlockSpec((_NODE_BLK, H), lambda i: (i, 0))] * 2,
        out_shape=[jax.ShapeDtypeStruct((N, H), jnp.float32)] * 2,
    )(h, agg, wuh, wua, bu, wmh)


def _final(h, bidx, arep, w1h, w1a, b1, w2, b2, w3, b3, w4, b4):
    return pl.pallas_call(
        _final_body,
        grid=(N // _NODE_BLK,),
        in_specs=[
            pl.BlockSpec((_NODE_BLK, H), lambda i: (i, 0)),
            pl.BlockSpec((_NODE_BLK, 1), lambda i: (i, 0)),
            _full((NB, 128)),
            _full((H, 256)), _full((128, 256)), _full((1, 256)),
            _full((256, 128)), _full((1, 128)),
            _full((128, 64)), _full((1, 64)),
            _full((64, 1)), _full((1, 1)),
        ],
        out_specs=pl.BlockSpec((_NODE_BLK, 1), lambda i: (i, 0)),
        out_shape=jax.ShapeDtypeStruct((N, 1), jnp.float32),
    )(h, bidx, arep, w1h, w1a, b1, w2, b2, w3, b3, w4, b4)


_SC_MESH = plsc.VectorSubcoreMesh(
    core_axis_name="c", subcore_axis_name="s", num_cores=NC, num_subcores=NS)


@functools.partial(
    pl.kernel,
    out_type=jax.ShapeDtypeStruct((NC, N, H), jnp.float32),
    mesh=_SC_MESH,
    compiler_params=pltpu.CompilerParams(use_tc_tiling_on_sc=False),
    scratch_types=[
        pltpu.VMEM((NG, CHUNK), jnp.int32),
        pltpu.VMEM((NG, CHUNK), jnp.int32),
    ]
    + [pltpu.VMEM((CHUNK, H), jnp.float32)] * NBUF
    + [
        pltpu.VMEM_SHARED((N, H), jnp.float32),
    ]
    + [pltpu.SemaphoreType.DMA] * (3 * NBUF),
)
def _sc_edge_step(hp_hbm, c_hbm, src_hbm, dst_hbm, out_hbm, src_v, dst_v,
                  *rest):
    bufs = rest[:NBUF]
    agg_s = rest[NBUF]
    csems = rest[NBUF + 1:2 * NBUF + 1]
    gsems = rest[2 * NBUF + 1:3 * NBUF + 1]
    ssems = rest[3 * NBUF + 1:4 * NBUF + 1]
    cid = lax.axis_index("c")
    sid = lax.axis_index("s")
    w = cid * NS + sid
    r0 = sid * RPT
    ebase = w * EPT

    def start_c(gg, b):
        pltpu.async_copy(c_hbm.at[pl.ds(ebase + gg * CHUNK, CHUNK)],
                         bufs[b], csems[b])

    def wait_c(b):
        pltpu.make_async_copy(c_hbm.at[pl.ds(0, CHUNK)],
                              bufs[b], csems[b]).wait()

    def start_gather(gg, b):
        pltpu.async_copy(hp_hbm.at[src_v.at[gg]], bufs[b], gsems[b], add=True)

    def wait_gather(b):
        pltpu.make_async_copy(hp_hbm.at[src_v.at[0]],
                              bufs[b], gsems[b]).wait()

    def start_scatter(gg, b):
        pltpu.async_copy(bufs[b], agg_s.at[dst_v.at[gg]], ssems[b], add=True)

    def wait_scatter(b):
        pltpu.make_async_copy(bufs[b], agg_s.at[dst_v.at[0]], ssems[b]).wait()

    pltpu.sync_copy(src_hbm.at[w], src_v)
    pltpu.sync_copy(dst_hbm.at[w], dst_v)

    zv = jnp.zeros((16,), jnp.float32)
    b0 = bufs[0]

    def zero_rows(i, _):
        for r in range(8):
            for j in range(H // 16):
                b0[i * 8 + r, pl.ds(j * 16, 16)] = zv
        return 0

    lax.fori_loop(0, CHUNK // 8, zero_rows, 0)

    for k in range(RPT // CHUNK):
        pltpu.sync_copy(b0, agg_s.at[pl.ds(r0 + k * CHUNK, CHUNK)])
    rem = RPT % CHUNK
    pltpu.sync_copy(b0.at[pl.ds(0, rem)],
                    agg_s.at[pl.ds(r0 + RPT - rem, rem)])

    for g0 in range(6):
        start_c(g0, g0)
    for g0 in range(3):
        wait_c(g0)
        start_gather(g0, g0)
    plsc.subcore_barrier()

    def super_iter(k8, _):
        for b in range(NBUF):
            gg = k8 * NBUF + b
            bp3 = (b + 3) % NBUF
            bp6 = (b + 6) % NBUF

            @pl.when(gg >= 2)
            def _(b_=bp6):
                wait_scatter(b_)

            @pl.when(gg + 6 < NG)
            def _(g_=gg, b_=bp6):
                start_c(g_ + 6, b_)

            @pl.when(gg + 3 < NG)
            def _(g_=gg, b_=bp3):
                wait_c(b_)
                start_gather(g_ + 3, b_)

            wait_gather(b)
            buf = bufs[b]

            def relu_rows(i, _):
                for r in range(8):
                    for j in range(H // 16):
                        sl = (i * 8 + r, pl.ds(j * 16, 16))
                        buf[sl] = jnp.maximum(buf[sl], 0.0)
                return 0

            lax.fori_loop(0, CHUNK // 8, relu_rows, 0)
            start_scatter(gg, b)
        return 0

    lax.fori_loop(0, NG // NBUF, super_iter, 0)
    wait_scatter((NG - 2) % NBUF)
    wait_scatter((NG - 1) % NBUF)
    plsc.subcore_barrier()
    pltpu.sync_copy(agg_s.at[pl.ds(r0, RPT)], out_hbm.at[cid, pl.ds(r0, RPT)])


def kernel(mol_a_reprs, node_features, edge_features, node_hiddens,
           edge_hiddens, Wn, bn, We, be, Wm, bm, Wu, bu,
           W1, b1, W2, b2, W3, b3, W4, b4, edge_indices, batch_indices):
    del node_hiddens, edge_hiddens

    wm_h, wm_e = Wm[:H], Wm[H:]
    wu_h, wu_a = Wu[:H], Wu[H:]
    w1_h, w1_a = W1[:H], W1[H:]
    bn2, be2, bm2, bu2 = (b.reshape(1, -1) for b in (bn, be, bm, bu))
    b12, b22, b32, b42 = (b.reshape(1, -1) for b in (b1, b2, b3, b4))

    ef_pad = jnp.pad(edge_features, ((0, EPAD - E), (0, 0)))
    ei_pad = jnp.pad(edge_indices, ((0, 0), (0, EPAD - E)))
    src3 = ei_pad[0].reshape(NW, NG, CHUNK)
    dst3 = ei_pad[1].reshape(NW, NG, CHUNK)

    c = _edge_pre(ef_pad, We, be2, wm_e, bm2)
    h, hp = _node_init(node_features, Wn, bn2, wm_h)
    for _ in range(8):
        agg = _sc_edge_step(hp, c, src3, dst3)
        h, hp = _update(h, agg, wu_h, wu_a, bu2, wm_h)

    out = _final(h, batch_indices.reshape(N, 1), mol_a_reprs,
                 w1_h, w1_a, b12, W2, b22, W3, b32, W4, b42)
    return out.astype(jnp.bool_)


# speedup vs baseline: 1.9503x; 1.9503x over previous
import functools

import jax
import jax.numpy as jnp
from jax import lax
from jax.experimental import pallas as pl
from jax.experimental.pallas import tpu as pltpu
from jax.experimental.pallas import tpu_sc as plsc

N = 10000
E = 320000
NB = 256
H = 64
NC, NS = 2, 16
NW = NC * NS
EPAD = 327680
EPT = EPAD // NW
CHUNK = 64
NG = EPT // CHUNK
NBUF = 4
NSUP = NG // NBUF
RPT = N // NS
NEG = -1.0e30

_NODE_BLK = 2000
_EDGE_BLK = 4096


def _node_init_body(nf, wn, bn, wmh, h_out, hp_out):
    hv = jnp.maximum(jnp.dot(nf[...], wn[...]) + bn[...], 0.0)
    h_out[...] = hv
    hp_out[...] = jnp.dot(hv, wmh[...])


def _edge_pre_body(ef, we, be, wme, bm, c_out):
    i = pl.program_id(0)
    ev = jnp.maximum(jnp.dot(ef[...], we[...]) + be[...], 0.0)
    cv = jnp.dot(ev, wme[...]) + bm[...]
    rows = i * _EDGE_BLK + lax.broadcasted_iota(jnp.int32, (_EDGE_BLK, 1), 0)
    c_out[...] = jnp.where(rows < E, cv, NEG)


def _update_body(h, agg, wuh, wua, bu, wmh, h_out, hp_out):
    a = agg[0] + agg[1]
    hv = jnp.maximum(
        jnp.dot(h[...], wuh[...]) + jnp.dot(a, wua[...]) + bu[...], 0.0)
    h_out[...] = hv
    hp_out[...] = jnp.dot(hv, wmh[...])


def _final_body(h, bidx, arep, w1h, w1a, b1, w2, b2, w3, b3, w4, b4, out):
    onehot = (bidx[...] == lax.broadcasted_iota(
        jnp.int32, (_NODE_BLK, NB), 1)).astype(jnp.float32)
    a = jnp.dot(onehot, arep[...])
    x = jnp.maximum(jnp.dot(h[...], w1h[...]) + jnp.dot(a, w1a[...]) + b1[...], 0.0)
    x = jnp.maximum(jnp.dot(x, w2[...]) + b2[...], 0.0)
    x = jnp.maximum(jnp.dot(x, w3[...]) + b3[...], 0.0)
    logit = jnp.dot(x, w4[...]) + b4[...]
    out[...] = (logit >= 0.0).astype(jnp.float32)


def _full(shape):
    return pl.BlockSpec(shape, lambda i: tuple(0 for _ in shape))


def _node_init(nf, wn, bn, wmh):
    return pl.pallas_call(
        _node_init_body,
        grid=(N // _NODE_BLK,),
        in_specs=[
            pl.BlockSpec((_NODE_BLK, 128), lambda i: (i, 0)),
            _full((128, H)), _full((1, H)), _full((H, H)),
        ],
        out_specs=[pl.BlockSpec((_NODE_BLK, H), lambda i: (i, 0))] * 2,
        out_shape=[jax.ShapeDtypeStruct((N, H), jnp.float32)] * 2,
    )(nf, wn, bn, wmh)


def _edge_pre(ef, we, be, wme, bm):
    return pl.pallas_call(
        _edge_pre_body,
        grid=(EPAD // _EDGE_BLK,),
        in_specs=[
            pl.BlockSpec((_EDGE_BLK, 16), lambda i: (i, 0)),
            _full((16, 32)), _full((1, 32)), _full((32, H)), _full((1, H)),
        ],
        out_specs=pl.BlockSpec((_EDGE_BLK, H), lambda i: (i, 0)),
        out_shape=jax.ShapeDtypeStruct((EPAD, H), jnp.float32),
    )(ef, we, be, wme, bm)


def _update(h, agg, wuh, wua, bu, wmh):
    return pl.pallas_call(
        _update_body,
        grid=(N // _NODE_BLK,),
        in_specs=[
            pl.BlockSpec((_NODE_BLK, H), lambda i: (i, 0)),
            pl.BlockSpec((NC, _NODE_BLK, H), lambda i: (0, i, 0)),
            _full((H, H)), _full((H, H)), _full((1, H)), _full((H, H)),
        ],
        out_specs=[pl.BlockSpec((_NODE_BLK, H), lambda i: (i, 0))] * 2,
        out_shape=[jax.ShapeDtypeStruct((N, H), jnp.float32)] * 2,
    )(h, agg, wuh, wua, bu, wmh)


def _final(h, bidx, arep, w1h, w1a, b1, w2, b2, w3, b3, w4, b4):
    return pl.pallas_call(
        _final_body,
        grid=(N // _NODE_BLK,),
        in_specs=[
            pl.BlockSpec((_NODE_BLK, H), lambda i: (i, 0)),
            pl.BlockSpec((_NODE_BLK, 1), lambda i: (i, 0)),
            _full((NB, 128)),
            _full((H, 256)), _full((128, 256)), _full((1, 256)),
            _full((256, 128)), _full((1, 128)),
            _full((128, 64)), _full((1, 64)),
            _full((64, 1)), _full((1, 1)),
        ],
        out_specs=pl.BlockSpec((_NODE_BLK, 1), lambda i: (i, 0)),
        out_shape=jax.ShapeDtypeStruct((N, 1), jnp.float32),
    )(h, bidx, arep, w1h, w1a, b1, w2, b2, w3, b3, w4, b4)


_SC_MESH = plsc.VectorSubcoreMesh(
    core_axis_name="c", subcore_axis_name="s", num_cores=NC, num_subcores=NS)


@functools.partial(
    pl.kernel,
    out_type=jax.ShapeDtypeStruct((NC, N, H), jnp.float32),
    mesh=_SC_MESH,
    compiler_params=pltpu.CompilerParams(use_tc_tiling_on_sc=False),
    scratch_types=[
        pltpu.VMEM((2, NBUF, CHUNK), jnp.int32),
        pltpu.VMEM((2, NBUF, CHUNK), jnp.int32),
    ]
    + [pltpu.VMEM((CHUNK, H), jnp.float32)] * NBUF
    + [
        pltpu.VMEM_SHARED((N, H), jnp.float32),
        pltpu.VMEM_SHARED((N, H), jnp.float32),
    ]
    + [pltpu.SemaphoreType.DMA] * (3 * NBUF + 2),
)
def _sc_edge_step(hp_hbm, c_hbm, src_hbm, dst_hbm, out_hbm, sidx, didx,
                  *rest):
    bufs = rest[:NBUF]
    hp_s = rest[NBUF]
    agg_s = rest[NBUF + 1]
    csems = rest[NBUF + 2:2 * NBUF + 2]
    gsems = rest[2 * NBUF + 2:3 * NBUF + 2]
    ssems = rest[3 * NBUF + 2:4 * NBUF + 2]
    isems = rest[4 * NBUF + 2:4 * NBUF + 4]
    cid = lax.axis_index("c")
    sid = lax.axis_index("s")
    w = cid * NS + sid
    r0 = sid * RPT
    ebase = w * EPT

    def start_c(gg, b):
        pltpu.async_copy(c_hbm.at[pl.ds(ebase + gg * CHUNK, CHUNK)],
                         bufs[b], csems[b])

    def wait_c(b):
        pltpu.make_async_copy(c_hbm.at[pl.ds(0, CHUNK)],
                              bufs[b], csems[b]).wait()

    def start_gather(p, row, b):
        pltpu.async_copy(hp_s.at[sidx.at[p, row]], bufs[b], gsems[b],
                         add=True)

    def wait_gather(b):
        pltpu.make_async_copy(hp_s.at[sidx.at[0, 0]],
                              bufs[b], gsems[b]).wait()

    def start_scatter(p, row, b):
        pltpu.async_copy(bufs[b], agg_s.at[didx.at[p, row]], ssems[b],
                         add=True)

    def wait_scatter(b):
        pltpu.make_async_copy(bufs[b], agg_s.at[didx.at[0, 0]],
                              ssems[b]).wait()

    def start_idx(k_next, par):
        pltpu.async_copy(src_hbm.at[w, k_next], sidx.at[par], isems[par])
        pltpu.async_copy(dst_hbm.at[w, k_next], didx.at[par], isems[par])

    def wait_idx(par):
        pltpu.make_async_copy(src_hbm.at[w, 0], sidx.at[par],
                              isems[par]).wait()
        pltpu.make_async_copy(dst_hbm.at[w, 0], didx.at[par],
                              isems[par]).wait()

    pltpu.sync_copy(src_hbm.at[w, 0], sidx.at[0])
    pltpu.sync_copy(dst_hbm.at[w, 0], didx.at[0])
    pltpu.sync_copy(hp_hbm.at[pl.ds(r0, RPT)], hp_s.at[pl.ds(r0, RPT)])

    zv = jnp.zeros((16,), jnp.float32)
    b0 = bufs[0]

    def zero_rows(i, _):
        for r in range(8):
            for j in range(H // 16):
                b0[i * 8 + r, pl.ds(j * 16, 16)] = zv
        return 0

    lax.fori_loop(0, CHUNK // 8, zero_rows, 0)
    for k in range(RPT // CHUNK):
        pltpu.sync_copy(b0, agg_s.at[pl.ds(r0 + k * CHUNK, CHUNK)])
    rem = RPT % CHUNK
    pltpu.sync_copy(b0.at[pl.ds(0, rem)],
                    agg_s.at[pl.ds(r0 + RPT - rem, rem)])

    start_c(0, 0)
    start_c(1, 1)
    wait_c(0)
    start_gather(0, 0, 0)
    plsc.subcore_barrier()

    def pair_iter(pi, _):
        for s in range(2):
            for b in range(NBUF):
                k = 2 * pi + s
                gg = k * NBUF + b
                bp1 = (b + 1) % NBUF
                bp2 = (b + 2) % NBUF

                @pl.when(gg >= 2)
                def _(b_=bp2):
                    wait_scatter(b_)

                @pl.when(gg + 2 < NG)
                def _(g_=gg, b_=bp2):
                    start_c(g_ + 2, b_)

                if b == 2:
                    @pl.when(k + 1 < NSUP)
                    def _(k_=k, s_=s):
                        start_idx(k_ + 1, 1 - s_)

                if b < NBUF - 1:
                    @pl.when(gg + 1 < NG)
                    def _(g_=gg, b_=bp1, s_=s):
                        wait_c(b_)
                        start_gather(s_, b_ + 0, b_)
                else:
                    @pl.when(gg + 1 < NG)
                    def _(g_=gg, b_=bp1, s_=s):
                        wait_c(b_)
                        wait_idx(1 - s_)
                        start_gather(1 - s_, 0, b_)

                wait_gather(b)
                buf = bufs[b]

                def relu_rows(i, _):
                    for r in range(8):
                        for j in range(H // 16):
                            sl = (i * 8 + r, pl.ds(j * 16, 16))
                            buf[sl] = jnp.maximum(buf[sl], 0.0)
                    return 0

                lax.fori_loop(0, CHUNK // 8, relu_rows, 0)
                start_scatter(s, b, b)
        return 0

    lax.fori_loop(0, NSUP // 2, pair_iter, 0)
    wait_scatter((NG - 2) % NBUF)
    wait_scatter((NG - 1) % NBUF)
    plsc.subcore_barrier()
    pltpu.sync_copy(agg_s.at[pl.ds(r0, RPT)], out_hbm.at[cid, pl.ds(r0, RPT)])


def kernel(mol_a_reprs, node_features, edge_features, node_hiddens,
           edge_hiddens, Wn, bn, We, be, Wm, bm, Wu, bu,
           W1, b1, W2, b2, W3, b3, W4, b4, edge_indices, batch_indices):
    del node_hiddens, edge_hiddens

    wm_h, wm_e = Wm[:H], Wm[H:]
    wu_h, wu_a = Wu[:H], Wu[H:]
    w1_h, w1_a = W1[:H], W1[H:]
    bn2, be2, bm2, bu2 = (b.reshape(1, -1) for b in (bn, be, bm, bu))
    b12, b22, b32, b42 = (b.reshape(1, -1) for b in (b1, b2, b3, b4))

    ef_pad = jnp.pad(edge_features, ((0, EPAD - E), (0, 0)))
    ei_pad = jnp.pad(edge_indices, ((0, 0), (0, EPAD - E)))
    src3 = ei_pad[0].reshape(NW, NSUP, NBUF, CHUNK)
    dst3 = ei_pad[1].reshape(NW, NSUP, NBUF, CHUNK)

    c = _edge_pre(ef_pad, We, be2, wm_e, bm2)
    h, hp = _node_init(node_features, Wn, bn2, wm_h)
    for _ in range(8):
        agg = _sc_edge_step(hp, c, src3, dst3)
        h, hp = _update(h, agg, wu_h, wu_a, bu2, wm_h)

    out = _final(h, batch_indices.reshape(N, 1), mol_a_reprs,
                 w1_h, w1_a, b12, W2, b22, W3, b32, W4, b42)
    return out.astype(jnp.bool_)


# speedup vs baseline: 1.9919x; 1.0213x over previous
import functools

import jax
import jax.numpy as jnp
from jax import lax
from jax.experimental import pallas as pl
from jax.experimental.pallas import tpu as pltpu
from jax.experimental.pallas import tpu_sc as plsc

N = 10000
E = 320000
NB = 256
H = 64
NC, NS = 2, 16
NW = NC * NS
EPAD = 327680
EPT = EPAD // NW
CHUNK = 64
NG = EPT // CHUNK
NBUF = 4
NSUP = NG // NBUF
RPT = N // NS
NEG = -1.0e30

_NODE_BLK = 2000
_EDGE_BLK = 4096


def _node_init_body(nf, wn, bn, wmh, h_out, hp_out):
    hv = jnp.maximum(jnp.dot(nf[...], wn[...]) + bn[...], 0.0)
    h_out[...] = hv
    hp_out[...] = jnp.dot(hv, wmh[...])


def _edge_pre_body(ef, we, be, wme, bm, c_out):
    i = pl.program_id(0)
    ev = jnp.maximum(jnp.dot(ef[...], we[...]) + be[...], 0.0)
    cv = jnp.dot(ev, wme[...]) + bm[...]
    rows = i * _EDGE_BLK + lax.broadcasted_iota(jnp.int32, (_EDGE_BLK, 1), 0)
    c_out[...] = jnp.where(rows < E, cv, NEG)


def _update_body(h, agg, wuh, wua, bu, wmh, h_out, hp_out):
    a = agg[0] + agg[1]
    hv = jnp.maximum(
        jnp.dot(h[...], wuh[...]) + jnp.dot(a, wua[...]) + bu[...], 0.0)
    h_out[...] = hv
    hp_out[...] = jnp.dot(hv, wmh[...])


def _final_body(h, bidx, arep, w1h, w1a, b1, w2, b2, w3, b3, w4, b4, out):
    onehot = (bidx[...] == lax.broadcasted_iota(
        jnp.int32, (_NODE_BLK, NB), 1)).astype(jnp.float32)
    a = jnp.dot(onehot, arep[...])
    x = jnp.maximum(jnp.dot(h[...], w1h[...]) + jnp.dot(a, w1a[...]) + b1[...], 0.0)
    x = jnp.maximum(jnp.dot(x, w2[...]) + b2[...], 0.0)
    x = jnp.maximum(jnp.dot(x, w3[...]) + b3[...], 0.0)
    logit = jnp.dot(x, w4[...]) + b4[...]
    out[...] = (logit >= 0.0).astype(jnp.float32)


def _full(shape):
    return pl.BlockSpec(shape, lambda i: tuple(0 for _ in shape))


def _node_init(nf, wn, bn, wmh):
    return pl.pallas_call(
        _node_init_body,
        grid=(N // _NODE_BLK,),
        in_specs=[
            pl.BlockSpec((_NODE_BLK, 128), lambda i: (i, 0)),
            _full((128, H)), _full((1, H)), _full((H, H)),
        ],
        out_specs=[pl.BlockSpec((_NODE_BLK, H), lambda i: (i, 0))] * 2,
        out_shape=[jax.ShapeDtypeStruct((N, H), jnp.float32)] * 2,
    )(nf, wn, bn, wmh)


def _edge_pre(ef, we, be, wme, bm):
    return pl.pallas_call(
        _edge_pre_body,
        grid=(EPAD // _EDGE_BLK,),
        in_specs=[
            pl.BlockSpec((_EDGE_BLK, 16),
                         lambda i: (jnp.minimum(i, E // _EDGE_BLK), 0)),
            _full((16, 32)), _full((1, 32)), _full((32, H)), _full((1, H)),
        ],
        out_specs=pl.BlockSpec((_EDGE_BLK, H), lambda i: (i, 0)),
        out_shape=jax.ShapeDtypeStruct((EPAD, H), jnp.float32),
    )(ef, we, be, wme, bm)


def _update(h, agg, wuh, wua, bu, wmh):
    return pl.pallas_call(
        _update_body,
        grid=(N // _NODE_BLK,),
        in_specs=[
            pl.BlockSpec((_NODE_BLK, H), lambda i: (i, 0)),
            pl.BlockSpec((NC, _NODE_BLK, H), lambda i: (0, i, 0)),
            _full((H, H)), _full((H, H)), _full((1, H)), _full((H, H)),
        ],
        out_specs=[pl.BlockSpec((_NODE_BLK, H), lambda i: (i, 0))] * 2,
        out_shape=[jax.ShapeDtypeStruct((N, H), jnp.float32)] * 2,
    )(h, agg, wuh, wua, bu, wmh)


def _final(h, bidx, arep, w1h, w1a, b1, w2, b2, w3, b3, w4, b4):
    return pl.pallas_call(
        _final_body,
        grid=(N // _NODE_BLK,),
        in_specs=[
            pl.BlockSpec((_NODE_BLK, H), lambda i: (i, 0)),
            pl.BlockSpec((_NODE_BLK, 1), lambda i: (i, 0)),
            _full((NB, 128)),
            _full((H, 256)), _full((128, 256)), _full((1, 256)),
            _full((256, 128)), _full((1, 128)),
            _full((128, 64)), _full((1, 64)),
            _full((64, 1)), _full((1, 1)),
        ],
        out_specs=pl.BlockSpec((_NODE_BLK, 1), lambda i: (i, 0)),
        out_shape=jax.ShapeDtypeStruct((N, 1), jnp.float32),
    )(h, bidx, arep, w1h, w1a, b1, w2, b2, w3, b3, w4, b4)


_SC_MESH = plsc.VectorSubcoreMesh(
    core_axis_name="c", subcore_axis_name="s", num_cores=NC, num_subcores=NS)


@functools.partial(
    pl.kernel,
    out_type=jax.ShapeDtypeStruct((NC, N, H), jnp.float32),
    mesh=_SC_MESH,
    compiler_params=pltpu.CompilerParams(use_tc_tiling_on_sc=False),
    scratch_types=[
        pltpu.VMEM((2, NBUF, CHUNK), jnp.int32),
        pltpu.VMEM((2, NBUF, CHUNK), jnp.int32),
    ]
    + [pltpu.VMEM((CHUNK, H), jnp.float32)] * NBUF
    + [
        pltpu.VMEM_SHARED((N, H), jnp.float32),
        pltpu.VMEM_SHARED((N, H), jnp.float32),
    ]
    + [pltpu.SemaphoreType.DMA] * (3 * NBUF + 2),
)
def _sc_edge_step(hp_hbm, c_hbm, src_hbm, dst_hbm, out_hbm, sidx, didx,
                  *rest):
    bufs = rest[:NBUF]
    hp_s = rest[NBUF]
    agg_s = rest[NBUF + 1]
    csems = rest[NBUF + 2:2 * NBUF + 2]
    gsems = rest[2 * NBUF + 2:3 * NBUF + 2]
    ssems = rest[3 * NBUF + 2:4 * NBUF + 2]
    isems = rest[4 * NBUF + 2:4 * NBUF + 4]
    cid = lax.axis_index("c")
    sid = lax.axis_index("s")
    w = cid * NS + sid
    r0 = sid * RPT
    ebase = w * EPT

    def start_c(gg, b):
        pltpu.async_copy(c_hbm.at[pl.ds(ebase + gg * CHUNK, CHUNK)],
                         bufs[b], csems[b])

    def wait_c(b):
        pltpu.make_async_copy(c_hbm.at[pl.ds(0, CHUNK)],
                              bufs[b], csems[b]).wait()

    def start_gather(p, row, b):
        pltpu.async_copy(hp_s.at[sidx.at[p, row]], bufs[b], gsems[b],
                         add=True)

    def wait_gather(b):
        pltpu.make_async_copy(hp_s.at[sidx.at[0, 0]],
                              bufs[b], gsems[b]).wait()

    def start_scatter(p, row, b):
        pltpu.async_copy(bufs[b], agg_s.at[didx.at[p, row]], ssems[b],
                         add=True)

    def wait_scatter(b):
        pltpu.make_async_copy(bufs[b], agg_s.at[didx.at[0, 0]],
                              ssems[b]).wait()

    def start_idx(k_next, par):
        pltpu.async_copy(src_hbm.at[w, k_next], sidx.at[par], isems[par])
        pltpu.async_copy(dst_hbm.at[w, k_next], didx.at[par], isems[par])

    def wait_idx(par):
        pltpu.make_async_copy(src_hbm.at[w, 0], sidx.at[par],
                              isems[par]).wait()
        pltpu.make_async_copy(dst_hbm.at[w, 0], didx.at[par],
                              isems[par]).wait()

    pltpu.sync_copy(src_hbm.at[w, 0], sidx.at[0])
    pltpu.sync_copy(dst_hbm.at[w, 0], didx.at[0])
    pltpu.sync_copy(hp_hbm.at[pl.ds(r0, RPT)], hp_s.at[pl.ds(r0, RPT)])

    zv = jnp.zeros((16,), jnp.float32)
    b0 = bufs[0]

    def zero_rows(i, _):
        for r in range(8):
            for j in range(H // 16):
                b0[i * 8 + r, pl.ds(j * 16, 16)] = zv
        return 0

    lax.fori_loop(0, CHUNK // 8, zero_rows, 0)
    for k in range(RPT // CHUNK):
        pltpu.sync_copy(b0, agg_s.at[pl.ds(r0 + k * CHUNK, CHUNK)])
    rem = RPT % CHUNK
    pltpu.sync_copy(b0.at[pl.ds(0, rem)],
                    agg_s.at[pl.ds(r0 + RPT - rem, rem)])

    start_c(0, 0)
    start_c(1, 1)
    wait_c(0)
    start_gather(0, 0, 0)
    plsc.subcore_barrier()

    def pair_iter(pi, _):
        for s in range(2):
            for b in range(NBUF):
                k = 2 * pi + s
                gg = k * NBUF + b
                bp1 = (b + 1) % NBUF
                bp2 = (b + 2) % NBUF

                @pl.when(gg >= 2)
                def _(b_=bp2):
                    wait_scatter(b_)

                @pl.when(gg + 2 < NG)
                def _(g_=gg, b_=bp2):
                    start_c(g_ + 2, b_)

                if b == 2:
                    @pl.when(k + 1 < NSUP)
                    def _(k_=k, s_=s):
                        start_idx(k_ + 1, 1 - s_)

                if b < NBUF - 1:
                    @pl.when(gg + 1 < NG)
                    def _(g_=gg, b_=bp1, s_=s):
                        wait_c(b_)
                        start_gather(s_, b_ + 0, b_)
                else:
                    @pl.when(gg + 1 < NG)
                    def _(g_=gg, b_=bp1, s_=s):
                        wait_c(b_)
                        wait_idx(1 - s_)
                        start_gather(1 - s_, 0, b_)

                wait_gather(b)
                buf = bufs[b]

                def relu_rows(i, _):
                    for r in range(8):
                        for j in range(H // 16):
                            sl = (i * 8 + r, pl.ds(j * 16, 16))
                            buf[sl] = jnp.maximum(buf[sl], 0.0)
                    return 0

                lax.fori_loop(0, CHUNK // 8, relu_rows, 0)
                start_scatter(s, b, b)
        return 0

    lax.fori_loop(0, NSUP // 2, pair_iter, 0)
    wait_scatter((NG - 2) % NBUF)
    wait_scatter((NG - 1) % NBUF)
    plsc.subcore_barrier()
    pltpu.sync_copy(agg_s.at[pl.ds(r0, RPT)], out_hbm.at[cid, pl.ds(r0, RPT)])


def kernel(mol_a_reprs, node_features, edge_features, node_hiddens,
           edge_hiddens, Wn, bn, We, be, Wm, bm, Wu, bu,
           W1, b1, W2, b2, W3, b3, W4, b4, edge_indices, batch_indices):
    del node_hiddens, edge_hiddens

    wm_h, wm_e = Wm[:H], Wm[H:]
    wu_h, wu_a = Wu[:H], Wu[H:]
    w1_h, w1_a = W1[:H], W1[H:]
    bn2, be2, bm2, bu2 = (b.reshape(1, -1) for b in (bn, be, bm, bu))
    b12, b22, b32, b42 = (b.reshape(1, -1) for b in (b1, b2, b3, b4))

    ei_pad = jnp.pad(edge_indices, ((0, 0), (0, EPAD - E)))
    src3 = ei_pad[0].reshape(NW, NSUP, NBUF, CHUNK)
    dst3 = ei_pad[1].reshape(NW, NSUP, NBUF, CHUNK)

    c = _edge_pre(edge_features, We, be2, wm_e, bm2)
    h, hp = _node_init(node_features, Wn, bn2, wm_h)
    for _ in range(8):
        agg = _sc_edge_step(hp, c, src3, dst3)
        h, hp = _update(h, agg, wu_h, wu_a, bu2, wm_h)

    out = _final(h, batch_indices.reshape(N, 1), mol_a_reprs,
                 w1_h, w1_a, b12, W2, b22, W3, b32, W4, b42)
    return out.astype(jnp.bool_)
